# baseline (speedup 1.0000x reference)
import functools

import jax
import jax.numpy as jnp
from jax import lax
from jax.experimental import pallas as pl
from jax.experimental.pallas import tpu as pltpu
from jax.experimental.pallas import tpu_sc as plsc

N, E, D = 10000, 320000, 128
NC, NS = 2, 16
NW = NC * NS
CHUNK = 80
NBUF = 4
NCHUNK = 128
NOUT = NCHUNK // NBUF
EPAD = NW * NCHUNK * CHUNK - E
NPAD = 10016
ROWS_PT = 640
ROWS_LAST = N - 15 * ROWS_PT

_sc_mesh = plsc.VectorSubcoreMesh(core_axis_name="c", subcore_axis_name="s")


@functools.partial(
    pl.kernel,
    out_type=jax.ShapeDtypeStruct((NC, N, D), jnp.float32),
    mesh=_sc_mesh,
    scratch_types=[
        pltpu.VMEM((4 * NBUF, CHUNK), jnp.int32),
        pltpu.VMEM((4 * NBUF, CHUNK), jnp.int32),
        [pltpu.VMEM((CHUNK, D), jnp.float32)] * NBUF,
        [pltpu.SemaphoreType.DMA] * NBUF,
        [pltpu.SemaphoreType.DMA] * NBUF,
        [pltpu.SemaphoreType.DMA] * 4,
        [pltpu.SemaphoreType.DMA] * 4,
        pltpu.VMEM_SHARED((NPAD, D), jnp.float32),
    ],
)
def _sc_aggregate(h_hbm, src_hbm, dst_hbm, zero_hbm, out_hbm,
                  src_v, dst_v, rows_v, gsem, ssem, xsem, ysem, agg_sh):
    cid = lax.axis_index("c")
    sid = lax.axis_index("s")
    wid = sid * NC + cid
    r0 = sid * ROWS_PT

    @pl.when(sid < NS - 1)
    def _():
        @pl.when(cid == 0)
        def _():
            pltpu.sync_copy(h_hbm.at[pl.ds(r0, ROWS_PT)],
                            agg_sh.at[pl.ds(r0, ROWS_PT)])

        @pl.when(cid == 1)
        def _():
            pltpu.sync_copy(zero_hbm.at[pl.ds(r0, ROWS_PT)],
                            agg_sh.at[pl.ds(r0, ROWS_PT)])

    @pl.when(sid == NS - 1)
    def _():
        @pl.when(cid == 0)
        def _():
            pltpu.sync_copy(h_hbm.at[pl.ds(r0, ROWS_LAST)],
                            agg_sh.at[pl.ds(r0, ROWS_LAST)])

        @pl.when(cid == 1)
        def _():
            pltpu.sync_copy(zero_hbm.at[pl.ds(r0, ROWS_LAST)],
                            agg_sh.at[pl.ds(r0, ROWS_LAST)])

    def fetch_idx(o, slot):
        pltpu.async_copy(src_hbm.at[wid, o],
                         src_v.at[pl.ds(slot * NBUF, NBUF)], xsem[slot])
        pltpu.async_copy(dst_hbm.at[wid, o],
                         dst_v.at[pl.ds(slot * NBUF, NBUF)], ysem[slot])

    def wait_idx(o, slot):
        pltpu.make_async_copy(src_hbm.at[wid, o],
                              src_v.at[pl.ds(slot * NBUF, NBUF)],
                              xsem[slot]).wait()
        pltpu.make_async_copy(dst_hbm.at[wid, o],
                              dst_v.at[pl.ds(slot * NBUF, NBUF)],
                              ysem[slot]).wait()

    def start_gather(row, buf):
        pltpu.async_copy(h_hbm.at[src_v.at[row]], rows_v[buf], gsem[buf])

    def wait_gather(row, buf):
        pltpu.make_async_copy(h_hbm.at[src_v.at[row]], rows_v[buf],
                              gsem[buf]).wait()

    def wait_scatter(buf):
        pltpu.make_async_copy(rows_v[buf], agg_sh.at[dst_v.at[0]],
                              ssem[buf]).wait()

    fetch_idx(0, 0)
    fetch_idx(1, 1)
    wait_idx(0, 0)
    for b in range(NBUF):
        start_gather(b, b)
    plsc.subcore_barrier()

    def outer(ss, carry):
        for oo in range(4):
            o = ss * 4 + oo

            @pl.when(o + 1 < NOUT)
            def _(oo=oo, o=o):
                wait_idx(o + 1, (oo + 1) % 4)

            for b in range(NBUF):
                row = oo * NBUF + b
                wait_gather(row, b)
                pltpu.async_copy(rows_v[b], agg_sh.at[dst_v.at[row]],
                                 ssem[b], add=True)
                nbuf2 = (b + 1) % NBUF
                if b < NBUF - 1:
                    @pl.when(o >= 1)
                    def _(row=row, nbuf2=nbuf2):
                        wait_scatter(nbuf2)
                        start_gather(row + 1, nbuf2)
                else:
                    @pl.when(o + 1 < NOUT)
                    def _(oo=oo, nbuf2=nbuf2):
                        wait_scatter(nbuf2)
                        start_gather(((oo + 1) % 4) * NBUF, nbuf2)

            @pl.when(o + 2 < NOUT)
            def _(oo=oo, o=o):
                fetch_idx(o + 2, (oo + 2) % 4)

        return carry

    lax.fori_loop(0, NOUT // 4, outer, 0)
    for b in range(NBUF):
        wait_scatter(b)
    plsc.subcore_barrier()

    @pl.when(sid < NS - 1)
    def _():
        pltpu.sync_copy(agg_sh.at[pl.ds(r0, ROWS_PT)],
                        out_hbm.at[cid, pl.ds(r0, ROWS_PT)])

    @pl.when(sid == NS - 1)
    def _():
        pltpu.sync_copy(agg_sh.at[pl.ds(r0, ROWS_LAST)],
                        out_hbm.at[cid, pl.ds(r0, ROWS_LAST)])


_BLK = 1000
_NB = N // _BLK


def _colsum_body(agg_ref, vn_ref, out_ref, acc_ref):
    step = pl.program_id(0)

    @pl.when(step == 0)
    def _():
        acc_ref[...] = jnp.zeros_like(acc_ref)

    x = jnp.squeeze(agg_ref[...], 0)
    acc_ref[...] += jnp.sum(x, axis=0, keepdims=True)

    @pl.when(step == pl.num_programs(0) - 1)
    def _():
        out_ref[...] = vn_ref[...] + acc_ref[...] * (1.0 / N)


def _matmul_body(a0_ref, a1_ref, vn_ref, w_ref, out_ref):
    x = jnp.squeeze(a0_ref[...], 0) + jnp.squeeze(a1_ref[...], 0)
    x = x + vn_ref[...]
    y = jnp.dot(x, w_ref[...], preferred_element_type=jnp.float32)
    out_ref[...] = jnp.maximum(y, 0.0)


def kernel(H, edge_index, W, virtual_node):
    src = jnp.concatenate(
        [edge_index[0], jnp.zeros((EPAD,), jnp.int32)]
    ).reshape(NW, NOUT, NBUF, CHUNK)
    dst = jnp.concatenate(
        [edge_index[1], jnp.full((EPAD,), N, jnp.int32)]
    ).reshape(NW, NOUT, NBUF, CHUNK)
    zeros = jnp.zeros((N, D), jnp.float32)

    agg = _sc_aggregate(H, src, dst, zeros)

    vn = pl.pallas_call(
        _colsum_body,
        grid=(NC * _NB,),
        in_specs=[
            pl.BlockSpec((1, _BLK, D), lambda i: (i // _NB, i % _NB, 0)),
            pl.BlockSpec((1, D), lambda i: (0, 0)),
        ],
        out_specs=pl.BlockSpec((1, D), lambda i: (0, 0)),
        out_shape=jax.ShapeDtypeStruct((1, D), jnp.float32),
        scratch_shapes=[pltpu.VMEM((1, D), jnp.float32)],
    )(agg, virtual_node)

    out = pl.pallas_call(
        _matmul_body,
        grid=(_NB,),
        in_specs=[
            pl.BlockSpec((1, _BLK, D), lambda i: (0, i, 0)),
            pl.BlockSpec((1, _BLK, D), lambda i: (1, i, 0)),
            pl.BlockSpec((1, D), lambda i: (0, 0)),
            pl.BlockSpec((D, D), lambda i: (0, 0)),
        ],
        out_specs=pl.BlockSpec((_BLK, D), lambda i: (i, 0)),
        out_shape=jax.ShapeDtypeStruct((N, D), jnp.float32),
    )(agg, agg, vn, W)
    return out


# speedup vs baseline: 3.3453x; 3.3453x over previous
import functools

import jax
import jax.numpy as jnp
from jax import lax
from jax.experimental import pallas as pl
from jax.experimental.pallas import tpu as pltpu
from jax.experimental.pallas import tpu_sc as plsc

N, E, D = 10000, 320000, 128
NC, NS = 2, 16
NW = NC * NS
CHUNK = 80
NCHUNK = 125
NPAD = 10000
ROWS_PT = 640
ROWS_LAST = N - 15 * ROWS_PT

_sc_mesh = plsc.VectorSubcoreMesh(core_axis_name="c", subcore_axis_name="s")


@functools.partial(
    pl.kernel,
    out_type=jax.ShapeDtypeStruct((NC, N, D), jnp.float32),
    mesh=_sc_mesh,
    scratch_types=[
        pltpu.VMEM((NCHUNK * CHUNK,), jnp.int32),
        pltpu.VMEM((NCHUNK, CHUNK), jnp.int32),
        [pltpu.VMEM((CHUNK, D), jnp.float32)] * 2,
        [pltpu.SemaphoreType.DMA] * 2,
        [pltpu.SemaphoreType.DMA] * 2,
        pltpu.SemaphoreType.DMA,
        pltpu.VMEM_SHARED((NPAD, D), jnp.float32),
    ],
)
def _sc_aggregate(h_hbm, src_hbm, dst_hbm, zero_hbm, out_hbm,
                  src_v, dst_v, rows_v, gsem, ssem, isem, agg_sh):
    cid = lax.axis_index("c")
    sid = lax.axis_index("s")
    wid = sid * NC + cid
    r0 = sid * ROWS_PT

    pltpu.async_copy(src_hbm.at[wid], src_v, isem)
    pltpu.make_async_copy(src_hbm.at[wid], src_v, isem).wait()
    pltpu.async_copy(dst_hbm.at[wid], dst_v, isem)
    pltpu.make_async_copy(dst_hbm.at[wid], dst_v, isem).wait()

    @pl.when(sid < NS - 1)
    def _():
        @pl.when(cid == 0)
        def _():
            pltpu.sync_copy(h_hbm.at[pl.ds(r0, ROWS_PT)],
                            agg_sh.at[pl.ds(r0, ROWS_PT)])

        @pl.when(cid == 1)
        def _():
            pltpu.sync_copy(zero_hbm.at[pl.ds(r0, ROWS_PT)],
                            agg_sh.at[pl.ds(r0, ROWS_PT)])

    @pl.when(sid == NS - 1)
    def _():
        @pl.when(cid == 0)
        def _():
            pltpu.sync_copy(h_hbm.at[pl.ds(r0, ROWS_LAST)],
                            agg_sh.at[pl.ds(r0, ROWS_LAST)])

        @pl.when(cid == 1)
        def _():
            pltpu.sync_copy(zero_hbm.at[pl.ds(r0, ROWS_LAST)],
                            agg_sh.at[pl.ds(r0, ROWS_LAST)])

    plsc.subcore_barrier()

    def start_gather(c, buf):
        pltpu.async_copy(h_hbm.at[src_v.at[pl.ds(c * CHUNK, CHUNK)]],
                         rows_v[buf], gsem[buf])

    def wait_gather(c, buf):
        pltpu.make_async_copy(h_hbm.at[src_v.at[pl.ds(c * CHUNK, CHUNK)]],
                              rows_v[buf], gsem[buf]).wait()

    def start_scatter(c, buf):
        pltpu.async_copy(rows_v[buf], agg_sh.at[dst_v.at[c]], ssem[buf],
                         add=True)

    def wait_scatter(buf):
        pltpu.make_async_copy(rows_v[buf], agg_sh.at[dst_v.at[0]],
                              ssem[buf]).wait()

    start_gather(0, 0)

    def body(i, carry):
        c = 2 * i

        @pl.when(i >= 1)
        def _():
            wait_scatter(1)

        start_gather(c + 1, 1)
        wait_gather(c, 0)
        start_scatter(c, 0)

        @pl.when(c + 2 < NCHUNK)
        def _():
            wait_scatter(0)
            start_gather(c + 2, 0)

        wait_gather(c + 1, 1)
        start_scatter(c + 1, 1)
        return carry

    lax.fori_loop(0, (NCHUNK - 1) // 2, body, 0)
    wait_gather(NCHUNK - 1, 0)
    start_scatter(NCHUNK - 1, 0)
    wait_scatter(0)
    wait_scatter(1)
    plsc.subcore_barrier()

    @pl.when(sid < NS - 1)
    def _():
        pltpu.sync_copy(agg_sh.at[pl.ds(r0, ROWS_PT)],
                        out_hbm.at[cid, pl.ds(r0, ROWS_PT)])

    @pl.when(sid == NS - 1)
    def _():
        pltpu.sync_copy(agg_sh.at[pl.ds(r0, ROWS_LAST)],
                        out_hbm.at[cid, pl.ds(r0, ROWS_LAST)])


_BLK = 1000
_NB = N // _BLK


def _colsum_body(agg_ref, vn_ref, out_ref, acc_ref):
    step = pl.program_id(0)

    @pl.when(step == 0)
    def _():
        acc_ref[...] = jnp.zeros_like(acc_ref)

    x = jnp.squeeze(agg_ref[...], 0)
    acc_ref[...] += jnp.sum(x, axis=0, keepdims=True)

    @pl.when(step == pl.num_programs(0) - 1)
    def _():
        out_ref[...] = vn_ref[...] + acc_ref[...] * (1.0 / N)


def _matmul_body(a0_ref, a1_ref, vn_ref, w_ref, out_ref):
    x = jnp.squeeze(a0_ref[...], 0) + jnp.squeeze(a1_ref[...], 0)
    x = x + vn_ref[...]
    y = jnp.dot(x, w_ref[...], preferred_element_type=jnp.float32)
    out_ref[...] = jnp.maximum(y, 0.0)


def kernel(H, edge_index, W, virtual_node):
    src = edge_index[0].reshape(NW, NCHUNK * CHUNK)
    dst = edge_index[1].reshape(NW, NCHUNK, CHUNK)
    zeros = jnp.zeros((N, D), jnp.float32)

    agg = _sc_aggregate(H, src, dst, zeros)

    vn = pl.pallas_call(
        _colsum_body,
        grid=(NC * _NB,),
        in_specs=[
            pl.BlockSpec((1, _BLK, D), lambda i: (i // _NB, i % _NB, 0)),
            pl.BlockSpec((1, D), lambda i: (0, 0)),
        ],
        out_specs=pl.BlockSpec((1, D), lambda i: (0, 0)),
        out_shape=jax.ShapeDtypeStruct((1, D), jnp.float32),
        scratch_shapes=[pltpu.VMEM((1, D), jnp.float32)],
    )(agg, virtual_node)

    out = pl.pallas_call(
        _matmul_body,
        grid=(_NB,),
        in_specs=[
            pl.BlockSpec((1, _BLK, D), lambda i: (0, i, 0)),
            pl.BlockSpec((1, _BLK, D), lambda i: (1, i, 0)),
            pl.BlockSpec((1, D), lambda i: (0, 0)),
            pl.BlockSpec((D, D), lambda i: (0, 0)),
        ],
        out_specs=pl.BlockSpec((_BLK, D), lambda i: (i, 0)),
        out_shape=jax.ShapeDtypeStruct((N, D), jnp.float32),
    )(agg, agg, vn, W)
    return out


# speedup vs baseline: 3.4017x; 1.0169x over previous
import functools

import jax
import jax.numpy as jnp
from jax import lax
from jax.experimental import pallas as pl
from jax.experimental.pallas import tpu as pltpu
from jax.experimental.pallas import tpu_sc as plsc

N, E, D = 10000, 320000, 128
NC, NS = 2, 16
NW = NC * NS
CHUNK = 80
NCHUNK = 125
NPAD = 10000
ROWS_PT = 640
ROWS_LAST = N - 15 * ROWS_PT

_sc_mesh = plsc.VectorSubcoreMesh(core_axis_name="c", subcore_axis_name="s")


@functools.partial(
    pl.kernel,
    out_type=jax.ShapeDtypeStruct((NC, N, D), jnp.float32),
    mesh=_sc_mesh,
    scratch_types=[
        pltpu.VMEM((NCHUNK * CHUNK,), jnp.int32),
        pltpu.VMEM((NCHUNK, CHUNK), jnp.int32),
        [pltpu.VMEM((CHUNK, D), jnp.float32)] * 2,
        [pltpu.SemaphoreType.DMA] * 2,
        [pltpu.SemaphoreType.DMA] * 2,
        [pltpu.SemaphoreType.DMA] * 2,
        pltpu.VMEM_SHARED((NPAD, D), jnp.float32),
    ],
)
def _sc_aggregate(h_hbm, src_hbm, dst_hbm, zero_hbm, out_hbm,
                  src_v, dst_v, rows_v, gsem, ssem, isem, agg_sh):
    cid = lax.axis_index("c")
    sid = lax.axis_index("s")
    wid = sid * NC + cid
    r0 = sid * ROWS_PT

    pltpu.async_copy(src_hbm.at[wid], src_v, isem[0])
    pltpu.async_copy(dst_hbm.at[wid], dst_v, isem[1])

    @pl.when(sid < NS - 1)
    def _():
        @pl.when(cid == 0)
        def _():
            pltpu.sync_copy(h_hbm.at[pl.ds(r0, ROWS_PT)],
                            agg_sh.at[pl.ds(r0, ROWS_PT)])

        @pl.when(cid == 1)
        def _():
            pltpu.sync_copy(zero_hbm.at[pl.ds(r0, ROWS_PT)],
                            agg_sh.at[pl.ds(r0, ROWS_PT)])

    @pl.when(sid == NS - 1)
    def _():
        @pl.when(cid == 0)
        def _():
            pltpu.sync_copy(h_hbm.at[pl.ds(r0, ROWS_LAST)],
                            agg_sh.at[pl.ds(r0, ROWS_LAST)])

        @pl.when(cid == 1)
        def _():
            pltpu.sync_copy(zero_hbm.at[pl.ds(r0, ROWS_LAST)],
                            agg_sh.at[pl.ds(r0, ROWS_LAST)])

    def start_gather(c, buf):
        pltpu.async_copy(h_hbm.at[src_v.at[pl.ds(c * CHUNK, CHUNK)]],
                         rows_v[buf], gsem[buf])

    def wait_gather(c, buf):
        pltpu.make_async_copy(h_hbm.at[src_v.at[pl.ds(c * CHUNK, CHUNK)]],
                              rows_v[buf], gsem[buf]).wait()

    def start_scatter(c, buf):
        pltpu.async_copy(rows_v[buf], agg_sh.at[dst_v.at[c]], ssem[buf],
                         add=True)

    def wait_scatter(buf):
        pltpu.make_async_copy(rows_v[buf], agg_sh.at[dst_v.at[0]],
                              ssem[buf]).wait()

    pltpu.make_async_copy(src_hbm.at[wid], src_v, isem[0]).wait()
    start_gather(0, 0)
    pltpu.make_async_copy(dst_hbm.at[wid], dst_v, isem[1]).wait()
    plsc.subcore_barrier()


    def body(i, carry):
        c = 2 * i

        @pl.when(i >= 1)
        def _():
            wait_scatter(1)

        start_gather(c + 1, 1)
        wait_gather(c, 0)
        start_scatter(c, 0)

        @pl.when(c + 2 < NCHUNK)
        def _():
            wait_scatter(0)
            start_gather(c + 2, 0)

        wait_gather(c + 1, 1)
        start_scatter(c + 1, 1)
        return carry

    lax.fori_loop(0, (NCHUNK - 1) // 2, body, 0)
    wait_gather(NCHUNK - 1, 0)
    start_scatter(NCHUNK - 1, 0)
    wait_scatter(0)
    wait_scatter(1)
    plsc.subcore_barrier()

    @pl.when(sid < NS - 1)
    def _():
        pltpu.sync_copy(agg_sh.at[pl.ds(r0, ROWS_PT)],
                        out_hbm.at[cid, pl.ds(r0, ROWS_PT)])

    @pl.when(sid == NS - 1)
    def _():
        pltpu.sync_copy(agg_sh.at[pl.ds(r0, ROWS_LAST)],
                        out_hbm.at[cid, pl.ds(r0, ROWS_LAST)])


_BLK = 1000
_NB = N // _BLK


def _colsum_body(agg_ref, vn_ref, out_ref, acc_ref):
    step = pl.program_id(0)

    @pl.when(step == 0)
    def _():
        acc_ref[...] = jnp.zeros_like(acc_ref)

    x = jnp.squeeze(agg_ref[...], 0)
    acc_ref[...] += jnp.sum(x, axis=0, keepdims=True)

    @pl.when(step == pl.num_programs(0) - 1)
    def _():
        out_ref[...] = vn_ref[...] + acc_ref[...] * (1.0 / N)


def _matmul_body(a0_ref, a1_ref, vn_ref, w_ref, out_ref):
    x = jnp.squeeze(a0_ref[...], 0) + jnp.squeeze(a1_ref[...], 0)
    x = x + vn_ref[...]
    y = jnp.dot(x, w_ref[...], preferred_element_type=jnp.float32)
    out_ref[...] = jnp.maximum(y, 0.0)


def kernel(H, edge_index, W, virtual_node):
    src = edge_index[0].reshape(NW, NCHUNK * CHUNK)
    dst = edge_index[1].reshape(NW, NCHUNK, CHUNK)
    zeros = jnp.zeros((N, D), jnp.float32)

    agg = _sc_aggregate(H, src, dst, zeros)

    vn = pl.pallas_call(
        _colsum_body,
        grid=(NC * _NB,),
        in_specs=[
            pl.BlockSpec((1, _BLK, D), lambda i: (i // _NB, i % _NB, 0)),
            pl.BlockSpec((1, D), lambda i: (0, 0)),
        ],
        out_specs=pl.BlockSpec((1, D), lambda i: (0, 0)),
        out_shape=jax.ShapeDtypeStruct((1, D), jnp.float32),
        scratch_shapes=[pltpu.VMEM((1, D), jnp.float32)],
    )(agg, virtual_node)

    out = pl.pallas_call(
        _matmul_body,
        grid=(_NB,),
        in_specs=[
            pl.BlockSpec((1, _BLK, D), lambda i: (0, i, 0)),
            pl.BlockSpec((1, _BLK, D), lambda i: (1, i, 0)),
            pl.BlockSpec((1, D), lambda i: (0, 0)),
            pl.BlockSpec((D, D), lambda i: (0, 0)),
        ],
        out_specs=pl.BlockSpec((_BLK, D), lambda i: (i, 0)),
        out_shape=jax.ShapeDtypeStruct((N, D), jnp.float32),
    )(agg, agg, vn, W)
    return out
